# Initial kernel scaffold; baseline (speedup 1.0000x reference)
import jax
import jax.numpy as jnp
from jax.experimental import pallas as pl


def kernel(x, n_id0, res_n_id0, edge_src0, edge_dst0, res_n_id1, edge_src1, edge_dst1, W1, att_src1, att_dst1, b1, W2, att_src2, att_dst2, b2):
    raise NotImplementedError("write your pallas kernel here")


# speedup vs baseline: 73.2639x; 73.2639x over previous
import functools

import jax
import jax.numpy as jnp
from jax import lax
from jax.experimental import pallas as pl
from jax.experimental.pallas import tpu as pltpu
from jax.experimental.pallas import tpu_sc as plsc

F32 = jnp.float32
I32 = jnp.int32

NWORK = 32
LEAK = 0.2
EPS = 1e-16


def _iota16():
    return lax.iota(I32, 16)


def _pat01():
    return (_iota16() >= 8).astype(I32)


def _lane8():
    return _iota16() & 7


def _perm(x, idx):
    return jnp.take_along_axis(
        x, idx, axis=0, mode=lax.GatherScatterMode.PROMISE_IN_BOUNDS)


def _mm_body(x_ref, w_ref, o_ref):
    o_ref[...] = jnp.dot(x_ref[...], w_ref[...])


def _tc_matmul(x, w, block_rows):
    n, d = x.shape
    dw, m = w.shape
    assert n % block_rows == 0
    return pl.pallas_call(
        _mm_body,
        grid=(n // block_rows,),
        in_specs=[
            pl.BlockSpec((block_rows, d), lambda i: (i, 0)),
            pl.BlockSpec((dw, m), lambda i: (0, 0)),
        ],
        out_specs=pl.BlockSpec((block_rows, m), lambda i: (i, 0)),
        out_shape=jax.ShapeDtypeStruct((n, m), F32),
    )(x, w)


def _combine1_body(acc_ref, b1_ref, w2e_ref, expmat_ref, s2_ref, a2_ref):
    p = acc_ref[...]
    t = p[0] + p[1]
    num = t[:, :64]
    den8 = t[:, 64:72]
    den = jnp.dot(den8, expmat_ref[...])
    z = num / (den + EPS) + b1_ref[...]
    h = jnp.where(z > 0, z, jnp.exp(jnp.minimum(z, 0.0)) - 1.0)
    hw = jnp.dot(h, w2e_ref[...])
    s2_ref[...] = hw[:, :64]
    a2_ref[...] = hw[:, 64:66]


def _tc_combine1(acc1, b1, w2e, expmat, n1):
    rows = 400
    grid = n1 // rows
    return pl.pallas_call(
        _combine1_body,
        grid=(grid,),
        in_specs=[
            pl.BlockSpec((2, rows, 72), lambda i: (0, i, 0)),
            pl.BlockSpec((1, 64), lambda i: (0, 0)),
            pl.BlockSpec((64, 66), lambda i: (0, 0)),
            pl.BlockSpec((8, 64), lambda i: (0, 0)),
        ],
        out_specs=[
            pl.BlockSpec((rows, 64), lambda i: (i, 0)),
            pl.BlockSpec((rows, 2), lambda i: (i, 0)),
        ],
        out_shape=[
            jax.ShapeDtypeStruct((n1, 64), F32),
            jax.ShapeDtypeStruct((n1, 2), F32),
        ],
    )(acc1, b1, w2e, expmat)


def _final_body(acc_ref, b2_ref, o_ref):
    t = acc_ref[0] + acc_ref[1]
    num = t[:2000, :64]
    den = t[:2000, 64:65]
    z = num / (den + EPS) + b2_ref[...]
    m = jnp.max(z, axis=1, keepdims=True)
    zm = z - m
    o_ref[...] = zm - jnp.log(jnp.sum(jnp.exp(zm), axis=1, keepdims=True))


def _tc_final(acc2, b2, n2, n2p):
    return pl.pallas_call(
        _final_body,
        grid=(1,),
        in_specs=[
            pl.BlockSpec((2, n2p, 72), lambda i: (0, 0, 0)),
            pl.BlockSpec((1, 64), lambda i: (0, 0)),
        ],
        out_specs=pl.BlockSpec((n2, 64), lambda i: (0, 0)),
        out_shape=jax.ShapeDtypeStruct((n2, 64), F32),
    )(acc2, b2)


def _sc_prep_body(nid_hbm, esrc_hbm, res_hbm, s_hbm, srcg_hbm, aldtab_hbm,
                  nid_v, idx_v, out_v, res_v, did_v, srow_v, aldout_v):
    c = lax.axis_index("c")
    s = lax.axis_index("s")
    w = s * 2 + c
    pltpu.sync_copy(nid_hbm, nid_v)

    epw = esrc_hbm.shape[0] // NWORK
    ch = 896
    nch = epw // ch

    def chunk(ci, carry):
        base = w * epw + ci * ch
        pltpu.sync_copy(esrc_hbm.at[pl.ds(base, ch)], idx_v)

        def vloop(vi, carry2):
            e = vi * 16
            src16 = idx_v[pl.ds(e, 16)]
            out_v[pl.ds(e, 16)] = plsc.load_gather(nid_v, [src16])
            return carry2

        lax.fori_loop(0, ch // 16, vloop, 0)
        pltpu.sync_copy(out_v, srcg_hbm.at[pl.ds(base, ch)])
        return carry

    lax.fori_loop(0, nch, chunk, 0)

    @pl.when(w < 25)
    def _():
        base = w * 400
        pltpu.sync_copy(res_hbm.at[pl.ds(base, 400)], res_v)

        def vloop2(vi, carry):
            e = vi * 16
            r16 = res_v[pl.ds(e, 16)]
            did_v[pl.ds(e, 16)] = plsc.load_gather(nid_v, [r16])
            return carry

        lax.fori_loop(0, 25, vloop2, 0)

        lane8 = _lane8()
        pat01 = _pat01()
        colpat = 72 + lane8

        def sub(ci, carry):
            pltpu.sync_copy(s_hbm.at[did_v.at[pl.ds(ci * 80, 80)]], srow_v)

            def ext(vi, carry2):
                rr = vi * 2
                v = plsc.load_gather(srow_v, [rr + pat01, colpat])
                plsc.store_scatter(aldout_v, [ci * 80 + rr + pat01, lane8], v)
                return carry2

            lax.fori_loop(0, 40, ext, 0)
            return carry

        lax.fori_loop(0, 5, sub, 0)
        pltpu.sync_copy(aldout_v, aldtab_hbm.at[pl.ds(base, 400)])


def _sc_prep(n_id0, esrc_pad, res_n_id0, s_mat):
    e0p = esrc_pad.shape[0]
    n = n_id0.shape[0]
    n1 = res_n_id0.shape[0]
    f = functools.partial(
        pl.kernel,
        out_type=[
            jax.ShapeDtypeStruct((e0p,), I32),
            jax.ShapeDtypeStruct((n1, 8), F32),
        ],
        mesh=plsc.VectorSubcoreMesh(core_axis_name="c", subcore_axis_name="s"),
        compiler_params=pltpu.CompilerParams(
            needs_layout_passes=False, use_tc_tiling_on_sc=False),
        scratch_types=[
            pltpu.VMEM((n,), I32),
            pltpu.VMEM((896,), I32),
            pltpu.VMEM((896,), I32),
            pltpu.VMEM((400,), I32),
            pltpu.VMEM((400,), I32),
            pltpu.VMEM((80, 80), F32),
            pltpu.VMEM((400, 8), F32),
        ],
    )
    return f(_sc_prep_body)(n_id0, esrc_pad, res_n_id0, s_mat)


_B1K = 128


def _sc_edge1_body(s_hbm, srcg_hbm, edst_hbm, aldtab_hbm, out_hbm,
                   sidx_v, didx_v, srow_v, prod_v, aldrow_v, zpad_v,
                   ald_sh, acc_sh):
    c = lax.axis_index("c")
    s = lax.axis_index("s")
    w = s * 2 + c

    n1p = out_hbm.shape[1]
    n1 = aldtab_hbm.shape[0]
    rows_per_tile = n1p // 16
    epw = srcg_hbm.shape[0] // NWORK
    nch = epw // _B1K

    lane8 = _lane8()
    pat01 = _pat01()
    zero16 = jnp.zeros((16,), F32)
    excol = 64 + lane8

    pltpu.sync_copy(
        aldtab_hbm.at[pl.ds(s * (n1 // 16), n1 // 16)],
        ald_sh.at[pl.ds(s * (n1 // 16), n1 // 16)],
    )

    @pl.when(s == 0)
    def _():
        for r in range(0, 16, 2):
            plsc.store_scatter(zpad_v, [r + pat01, lane8], zero16)
        pltpu.sync_copy(zpad_v, ald_sh.at[pl.ds(n1, n1p - n1)])

    def zrow(r, carry):
        for i in range(4):
            prod_v[r, pl.ds(i * 16, 16)] = zero16
        plsc.store_scatter(prod_v, [r + pat01, excol], zero16)
        return carry

    lax.fori_loop(0, _B1K, zrow, 0)

    nfull = rows_per_tile // _B1K
    rem = rows_per_tile - nfull * _B1K

    def zacc(k, carry):
        pltpu.sync_copy(prod_v, acc_sh.at[pl.ds(s * rows_per_tile + k * _B1K, _B1K)])
        return carry

    lax.fori_loop(0, nfull, zacc, 0)
    if rem:
        pltpu.sync_copy(
            prod_v.at[pl.ds(0, rem)],
            acc_sh.at[pl.ds(s * rows_per_tile + nfull * _B1K, rem)],
        )
    plsc.subcore_barrier()

    def chunk(ci, carry):
        base = w * epw + ci * _B1K
        pltpu.sync_copy(srcg_hbm.at[pl.ds(base, _B1K)], sidx_v)
        pltpu.sync_copy(edst_hbm.at[pl.ds(base, _B1K)], didx_v)
        pltpu.sync_copy(s_hbm.at[sidx_v], srow_v)
        pltpu.sync_copy(ald_sh.at[didx_v], aldrow_v)

        def pair(j2, carry2):
            j = j2 * 2
            als = plsc.load_gather(srow_v, [j + pat01, 64 + lane8])
            ald = plsc.load_gather(aldrow_v, [j + pat01, lane8])
            a = als + ald
            ex = jnp.exp(jnp.maximum(a, LEAK * a))
            plsc.store_scatter(prod_v, [j + pat01, excol], ex)
            for i in range(4):
                coef0 = _perm(ex, pat01 + 2 * i)
                prod_v[j, pl.ds(i * 16, 16)] = (
                    srow_v[j, pl.ds(i * 16, 16)] * coef0
                )
                coef1 = _perm(ex, 8 + pat01 + 2 * i)
                prod_v[j + 1, pl.ds(i * 16, 16)] = (
                    srow_v[j + 1, pl.ds(i * 16, 16)] * coef1
                )
            return carry2

        lax.fori_loop(0, _B1K // 2, pair, 0)
        pltpu.sync_copy(prod_v, acc_sh.at[didx_v], add=True)
        return carry

    lax.fori_loop(0, nch, chunk, 0)

    plsc.subcore_barrier()
    pltpu.sync_copy(
        acc_sh.at[pl.ds(s * rows_per_tile, rows_per_tile)],
        out_hbm.at[c].at[pl.ds(s * rows_per_tile, rows_per_tile)],
    )


def _sc_edge1(s_mat, src_glob, edst_pad, ald_tab, n1p):
    f = functools.partial(
        pl.kernel,
        out_type=jax.ShapeDtypeStruct((2, n1p, 72), F32),
        mesh=plsc.VectorSubcoreMesh(core_axis_name="c", subcore_axis_name="s"),
        compiler_params=pltpu.CompilerParams(
            needs_layout_passes=False, use_tc_tiling_on_sc=False),
        scratch_types=[
            pltpu.VMEM((_B1K,), I32),
            pltpu.VMEM((_B1K,), I32),
            pltpu.VMEM((_B1K, 80), F32),
            pltpu.VMEM((_B1K, 72), F32),
            pltpu.VMEM((_B1K, 8), F32),
            pltpu.VMEM((16, 8), F32),
            pltpu.VMEM_SHARED((n1p, 8), F32),
            pltpu.VMEM_SHARED((n1p, 72), F32),
        ],
    )
    return f(_sc_edge1_body)(s_mat, src_glob, edst_pad, ald_tab)


def _sc_edge2_body(s2_hbm, a2_hbm, res1_hbm, esrc_hbm, edst_hbm, out_hbm,
                   a2_v, res_v, ald2_v, sidx_v, didx_v, srow_v, prod_v, acc_sh):
    c = lax.axis_index("c")
    s = lax.axis_index("s")
    w = s * 2 + c

    n2p = out_hbm.shape[1]
    rows_per_tile = n2p // 16
    epw = esrc_hbm.shape[0] // NWORK
    nch = epw // _B1K

    iota = _iota16()
    lane8 = _lane8()
    pat01 = _pat01()
    zero16 = jnp.zeros((16,), F32)
    zeros_i = iota * 0
    ones_i = zeros_i + 1
    col64 = zeros_i + 64
    excol = 64 + lane8

    pltpu.sync_copy(a2_hbm, a2_v)
    pltpu.sync_copy(res1_hbm, res_v)

    def bald(vi, carry):
        e = vi * 16
        r16 = res_v[pl.ds(e, 16)]
        ald2_v[pl.ds(e, 16)] = plsc.load_gather(a2_v, [r16, ones_i])
        return carry

    lax.fori_loop(0, res_v.shape[0] // 16, bald, 0)

    def zrow(r, carry):
        for i in range(4):
            prod_v[r, pl.ds(i * 16, 16)] = zero16
        plsc.store_scatter(prod_v, [r + pat01, excol], zero16)
        return carry

    lax.fori_loop(0, _B1K, zrow, 0)
    pltpu.sync_copy(
        prod_v.at[pl.ds(0, rows_per_tile)],
        acc_sh.at[pl.ds(s * rows_per_tile, rows_per_tile)],
    )
    plsc.subcore_barrier()

    dclamp = ald2_v.shape[0] - 1

    def chunk(ci, carry):
        base = w * epw + ci * _B1K
        pltpu.sync_copy(esrc_hbm.at[pl.ds(base, _B1K)], sidx_v)
        pltpu.sync_copy(edst_hbm.at[pl.ds(base, _B1K)], didx_v)
        pltpu.sync_copy(s2_hbm.at[sidx_v], srow_v)

        def grp(g, carry2):
            e0 = g * 16
            s16 = sidx_v[pl.ds(e0, 16)]
            d16 = didx_v[pl.ds(e0, 16)]
            d16 = jnp.minimum(d16, dclamp)
            als = plsc.load_gather(a2_v, [s16, zeros_i])
            ald = plsc.load_gather(ald2_v, [d16])
            a = als + ald
            ex = jnp.exp(jnp.maximum(a, LEAK * a))
            plsc.store_scatter(prod_v, [e0 + iota, col64], ex)
            for e in range(16):
                coef = _perm(ex, zeros_i + e)
                for i in range(4):
                    prod_v[e0 + e, pl.ds(i * 16, 16)] = (
                        srow_v[e0 + e, pl.ds(i * 16, 16)] * coef
                    )
            return carry2

        lax.fori_loop(0, _B1K // 16, grp, 0)
        pltpu.sync_copy(prod_v, acc_sh.at[didx_v], add=True)
        return carry

    lax.fori_loop(0, nch, chunk, 0)

    plsc.subcore_barrier()
    pltpu.sync_copy(
        acc_sh.at[pl.ds(s * rows_per_tile, rows_per_tile)],
        out_hbm.at[c].at[pl.ds(s * rows_per_tile, rows_per_tile)],
    )


def _sc_edge2(s2_mat, a2, res_n_id1, esrc_pad, edst_pad, n2p):
    f = functools.partial(
        pl.kernel,
        out_type=jax.ShapeDtypeStruct((2, n2p, 72), F32),
        mesh=plsc.VectorSubcoreMesh(core_axis_name="c", subcore_axis_name="s"),
        compiler_params=pltpu.CompilerParams(
            needs_layout_passes=False, use_tc_tiling_on_sc=False),
        scratch_types=[
            pltpu.VMEM(a2.shape, F32),
            pltpu.VMEM(res_n_id1.shape, I32),
            pltpu.VMEM((res_n_id1.shape[0],), F32),
            pltpu.VMEM((_B1K,), I32),
            pltpu.VMEM((_B1K,), I32),
            pltpu.VMEM((_B1K, 64), F32),
            pltpu.VMEM((_B1K, 72), F32),
            pltpu.VMEM_SHARED((n2p, 72), F32),
        ],
    )
    return f(_sc_edge2_body)(s2_mat, a2, res_n_id1, esrc_pad, edst_pad)


def kernel(x, n_id0, res_n_id0, edge_src0, edge_dst0, res_n_id1, edge_src1,
           edge_dst1, W1, att_src1, att_dst1, b1, W2, att_src2, att_dst2, b2):
    n, d = x.shape
    n1 = res_n_id0.shape[0]
    n2 = res_n_id1.shape[0]
    e0 = edge_src0.shape[0]
    e1 = edge_src1.shape[0]
    h1, c1 = att_src1.shape

    e0p = ((e0 + NWORK * _B1K - 1) // (NWORK * _B1K)) * (NWORK * _B1K)
    e1p = ((e1 + NWORK * _B1K - 1) // (NWORK * _B1K)) * (NWORK * _B1K)
    n1p = n1 + 16
    n2p = n2 + 16

    w1r = W1.reshape(d, h1, c1)
    ws1 = (w1r * att_src1[None]).sum(-1)
    wd1 = (w1r * att_dst1[None]).sum(-1)
    w1e = jnp.concatenate([W1, ws1, wd1], axis=1)

    h2, c2 = att_src2.shape
    w2r = W2.reshape(h1 * c1, h2, c2)
    ws2 = (w2r * att_src2[None]).sum(-1)
    wd2 = (w2r * att_dst2[None]).sum(-1)
    w2e = jnp.concatenate([W2, ws2, wd2], axis=1)

    expmat = jnp.repeat(jnp.eye(h1, dtype=F32), c1, axis=1)

    pad0 = e0p - e0
    pad1 = e1p - e1
    esrc0_p = jnp.concatenate([edge_src0, jnp.zeros((pad0,), I32)])
    edst0_p = jnp.concatenate(
        [edge_dst0, n1 + (jnp.arange(pad0, dtype=I32) % 16)])
    esrc1_p = jnp.concatenate([edge_src1, jnp.zeros((pad1,), I32)])
    edst1_p = jnp.concatenate(
        [edge_dst1, n2 + (jnp.arange(pad1, dtype=I32) % 16)])

    s_mat = _tc_matmul(x, w1e, 1000)

    src_glob, ald_tab = _sc_prep(n_id0, esrc0_p, res_n_id0, s_mat)

    acc1 = _sc_edge1(s_mat, src_glob, edst0_p, ald_tab, n1p)

    s2_mat, a2 = _tc_combine1(acc1, b1.reshape(1, 64), w2e, expmat, n1)

    acc2 = _sc_edge2(s2_mat, a2, res_n_id1, esrc1_p, edst1_p, n2p)

    return _tc_final(acc2, b2.reshape(1, 64), n2, n2p)


# speedup vs baseline: 185.6708x; 2.5343x over previous
import functools

import jax
import jax.numpy as jnp
from jax import lax
from jax.experimental import pallas as pl
from jax.experimental.pallas import tpu as pltpu
from jax.experimental.pallas import tpu_sc as plsc

F32 = jnp.float32
I32 = jnp.int32

NWORK = 32
LEAK = 0.2
EPS = 1e-16


def _iota16():
    return lax.iota(I32, 16)


def _pat01():
    return (_iota16() >= 8).astype(I32)


def _lane8():
    return _iota16() & 7


def _perm(x, idx):
    return jnp.take_along_axis(
        x, idx, axis=0, mode=lax.GatherScatterMode.PROMISE_IN_BOUNDS)


def _mm_body(x_ref, w_ref, o_ref):
    o_ref[...] = jnp.dot(x_ref[...], w_ref[...])


def _tc_matmul(x, w, block_rows):
    n, d = x.shape
    dw, m = w.shape
    assert n % block_rows == 0
    return pl.pallas_call(
        _mm_body,
        grid=(n // block_rows,),
        in_specs=[
            pl.BlockSpec((block_rows, d), lambda i: (i, 0)),
            pl.BlockSpec((dw, m), lambda i: (0, 0)),
        ],
        out_specs=pl.BlockSpec((block_rows, m), lambda i: (i, 0)),
        out_shape=jax.ShapeDtypeStruct((n, m), F32),
    )(x, w)


def _combine1_body(acc_ref, b1_ref, w2e_ref, expmat_ref, s2_ref, a2_ref):
    p = acc_ref[...]
    t = p[0] + p[1]
    num = t[:, :64]
    den8 = t[:, 64:72]
    den = jnp.dot(den8, expmat_ref[...])
    z = num / (den + EPS) + b1_ref[...]
    h = jnp.where(z > 0, z, jnp.exp(jnp.minimum(z, 0.0)) - 1.0)
    hw = jnp.dot(h, w2e_ref[...])
    s2_ref[...] = hw[:, :64]
    a2_ref[...] = hw[:, 64:66]


def _tc_combine1(acc1, b1, w2e, expmat, n1):
    rows = 400
    grid = n1 // rows
    return pl.pallas_call(
        _combine1_body,
        grid=(grid,),
        in_specs=[
            pl.BlockSpec((2, rows, 72), lambda i: (0, i, 0)),
            pl.BlockSpec((1, 64), lambda i: (0, 0)),
            pl.BlockSpec((64, 66), lambda i: (0, 0)),
            pl.BlockSpec((8, 64), lambda i: (0, 0)),
        ],
        out_specs=[
            pl.BlockSpec((rows, 64), lambda i: (i, 0)),
            pl.BlockSpec((rows, 2), lambda i: (i, 0)),
        ],
        out_shape=[
            jax.ShapeDtypeStruct((n1, 64), F32),
            jax.ShapeDtypeStruct((n1, 2), F32),
        ],
    )(acc1, b1, w2e, expmat)


def _final_body(acc_ref, b2_ref, o_ref):
    t = acc_ref[0] + acc_ref[1]
    num = t[:2000, :64]
    den = t[:2000, 64:65]
    z = num / (den + EPS) + b2_ref[...]
    m = jnp.max(z, axis=1, keepdims=True)
    zm = z - m
    o_ref[...] = zm - jnp.log(jnp.sum(jnp.exp(zm), axis=1, keepdims=True))


def _tc_final(acc2, b2, n2, n2p):
    return pl.pallas_call(
        _final_body,
        grid=(1,),
        in_specs=[
            pl.BlockSpec((2, n2p, 72), lambda i: (0, 0, 0)),
            pl.BlockSpec((1, 64), lambda i: (0, 0)),
        ],
        out_specs=pl.BlockSpec((n2, 64), lambda i: (0, 0)),
        out_shape=jax.ShapeDtypeStruct((n2, 64), F32),
    )(acc2, b2)


def _sc_prep_body(nid_hbm, esrc_hbm, res_hbm, s_hbm, srcg_hbm, aldtab_hbm,
                  nid_v, idx_v, out_v, res_v, did_v, srow_v, aldout_v):
    c = lax.axis_index("c")
    s = lax.axis_index("s")
    w = s * 2 + c
    pltpu.sync_copy(nid_hbm, nid_v)

    epw = esrc_hbm.shape[0] // NWORK
    ch = 896
    nch = epw // ch

    def chunk(ci, carry):
        base = w * epw + ci * ch
        pltpu.sync_copy(esrc_hbm.at[pl.ds(base, ch)], idx_v)

        def vloop(vi, carry2):
            e = vi * 16
            src16 = idx_v[pl.ds(e, 16)]
            out_v[pl.ds(e, 16)] = plsc.load_gather(nid_v, [src16])
            return carry2

        lax.fori_loop(0, ch // 16, vloop, 0)
        pltpu.sync_copy(out_v, srcg_hbm.at[pl.ds(base, ch)])
        return carry

    lax.fori_loop(0, nch, chunk, 0)

    @pl.when(w < 25)
    def _():
        base = w * 400
        pltpu.sync_copy(res_hbm.at[pl.ds(base, 400)], res_v)

        def vloop2(vi, carry):
            e = vi * 16
            r16 = res_v[pl.ds(e, 16)]
            did_v[pl.ds(e, 16)] = plsc.load_gather(nid_v, [r16])
            return carry

        lax.fori_loop(0, 25, vloop2, 0)

        lane8 = _lane8()
        pat01 = _pat01()
        colpat = 72 + lane8

        def sub(ci, carry):
            pltpu.sync_copy(s_hbm.at[did_v.at[pl.ds(ci * 80, 80)]], srow_v)

            def ext(vi, carry2):
                rr = vi * 2
                v = plsc.load_gather(srow_v, [rr + pat01, colpat])
                plsc.store_scatter(aldout_v, [ci * 80 + rr + pat01, lane8], v)
                return carry2

            lax.fori_loop(0, 40, ext, 0)
            return carry

        lax.fori_loop(0, 5, sub, 0)
        pltpu.sync_copy(aldout_v, aldtab_hbm.at[pl.ds(base, 400)])


def _sc_prep(n_id0, esrc_pad, res_n_id0, s_mat):
    e0p = esrc_pad.shape[0]
    n = n_id0.shape[0]
    n1 = res_n_id0.shape[0]
    f = functools.partial(
        pl.kernel,
        out_type=[
            jax.ShapeDtypeStruct((e0p,), I32),
            jax.ShapeDtypeStruct((n1, 8), F32),
        ],
        mesh=plsc.VectorSubcoreMesh(core_axis_name="c", subcore_axis_name="s"),
        compiler_params=pltpu.CompilerParams(
            needs_layout_passes=False, use_tc_tiling_on_sc=False),
        scratch_types=[
            pltpu.VMEM((n,), I32),
            pltpu.VMEM((896,), I32),
            pltpu.VMEM((896,), I32),
            pltpu.VMEM((400,), I32),
            pltpu.VMEM((400,), I32),
            pltpu.VMEM((80, 80), F32),
            pltpu.VMEM((400, 8), F32),
        ],
    )
    return f(_sc_prep_body)(n_id0, esrc_pad, res_n_id0, s_mat)


_B1K = 128


def _edge_pipeline(nch, idx_descs, gather_descs, scatter_desc, compute):
    for b in (0, 1):
        for d in idx_descs(b, b):
            d.start()
    for d in idx_descs(0, 0):
        d.wait()
    for d in gather_descs(0, 0):
        d.start()

    def group(g, carry):
        for b in range(4):
            ci = g * 4 + b
            s2 = b % 2
            rn = (b + 1) % 4
            rp2 = (b + 2) % 4
            for d in gather_descs(s2, b):
                d.wait()

            @pl.when(ci + 1 < nch)
            def _():
                for d in idx_descs(rn, ci + 1):
                    d.wait()
                for d in gather_descs(1 - s2, rn):
                    d.start()

            @pl.when(ci >= 2)
            def _():
                scatter_desc(s2, rp2).wait()

            @pl.when(ci + 2 < nch)
            def _():
                for d in idx_descs(rp2, ci + 2):
                    d.start()

            compute(s2, b)
            scatter_desc(s2, b).start(add=True)
        return carry

    lax.fori_loop(0, nch // 4, group, 0)
    scatter_desc(0, 2).wait()
    scatter_desc(1, 3).wait()


def _sc_edge1_body(s_hbm, srcg_hbm, edst_hbm, aldtab_hbm, out_hbm,
                   sidx0, sidx1, sidx2, sidx3, didx0, didx1, didx2, didx3,
                   srow0, srow1, prod0, prod1, aldrow0, aldrow1, zpad_v,
                   isem0, isem1, isem2, isem3, gsem0, gsem1, asem0, asem1,
                   ssem0, ssem1, ald_sh, acc_sh):
    c = lax.axis_index("c")
    s = lax.axis_index("s")
    w = s * 2 + c
    sidx = [sidx0, sidx1, sidx2, sidx3]
    didx = [didx0, didx1, didx2, didx3]
    srow = [srow0, srow1]
    prod = [prod0, prod1]
    aldrow = [aldrow0, aldrow1]
    isem = [isem0, isem1, isem2, isem3]
    gsem = [gsem0, gsem1]
    asem = [asem0, asem1]
    ssem = [ssem0, ssem1]

    n1p = out_hbm.shape[1]
    n1 = aldtab_hbm.shape[0]
    rows_per_tile = n1p // 16
    epw = srcg_hbm.shape[0] // NWORK
    nch = epw // _B1K

    lane8 = _lane8()
    pat01 = _pat01()
    zero16 = jnp.zeros((16,), F32)
    excol = 64 + lane8
    prod_v = prod0

    pltpu.sync_copy(
        aldtab_hbm.at[pl.ds(s * (n1 // 16), n1 // 16)],
        ald_sh.at[pl.ds(s * (n1 // 16), n1 // 16)],
    )

    @pl.when(s == 0)
    def _():
        for r in range(0, 16, 2):
            plsc.store_scatter(zpad_v, [r + pat01, lane8], zero16)
        pltpu.sync_copy(zpad_v, ald_sh.at[pl.ds(n1, n1p - n1)])

    def zrow(r, carry):
        for i in range(4):
            prod_v[r, pl.ds(i * 16, 16)] = zero16
        plsc.store_scatter(prod_v, [r + pat01, excol], zero16)
        return carry

    lax.fori_loop(0, _B1K, zrow, 0)

    nfull = rows_per_tile // _B1K
    rem = rows_per_tile - nfull * _B1K

    def zacc(k, carry):
        pltpu.sync_copy(prod_v, acc_sh.at[pl.ds(s * rows_per_tile + k * _B1K, _B1K)])
        return carry

    lax.fori_loop(0, nfull, zacc, 0)
    if rem:
        pltpu.sync_copy(
            prod_v.at[pl.ds(0, rem)],
            acc_sh.at[pl.ds(s * rows_per_tile + nfull * _B1K, rem)],
        )
    plsc.subcore_barrier()

    def idx_descs(r, ci):
        base = w * epw + ci * _B1K
        return (
            pltpu.make_async_copy(
                srcg_hbm.at[pl.ds(base, _B1K)], sidx[r], isem[r]),
            pltpu.make_async_copy(
                edst_hbm.at[pl.ds(base, _B1K)], didx[r], isem[r]),
        )

    def gather_descs(s2, r):
        return (
            pltpu.make_async_copy(s_hbm.at[sidx[r]], srow[s2], gsem[s2]),
            pltpu.make_async_copy(ald_sh.at[didx[r]], aldrow[s2], asem[s2]),
        )

    def scatter_desc(s2, r):
        return pltpu.make_async_copy(prod[s2], acc_sh.at[didx[r]], ssem[s2])

    def compute(s2, b):
        srow_v = srow[s2]
        aldrow_v = aldrow[s2]
        prod_w = prod[s2]

        @plsc.parallel_loop(0, _B1K // 2, unroll=4)
        def _(j2):
            j = j2 * 2
            als = plsc.load_gather(srow_v, [j + pat01, 64 + lane8])
            ald = plsc.load_gather(aldrow_v, [j + pat01, lane8])
            a = als + ald
            ex = jnp.exp(jnp.maximum(a, LEAK * a))
            plsc.store_scatter(prod_w, [j + pat01, excol], ex)
            for i in range(4):
                coef0 = _perm(ex, pat01 + 2 * i)
                prod_w[j, pl.ds(i * 16, 16)] = (
                    srow_v[j, pl.ds(i * 16, 16)] * coef0
                )
                coef1 = _perm(ex, 8 + pat01 + 2 * i)
                prod_w[j + 1, pl.ds(i * 16, 16)] = (
                    srow_v[j + 1, pl.ds(i * 16, 16)] * coef1
                )

    _edge_pipeline(nch, idx_descs, gather_descs, scatter_desc, compute)

    plsc.subcore_barrier()
    pltpu.sync_copy(
        acc_sh.at[pl.ds(s * rows_per_tile, rows_per_tile)],
        out_hbm.at[c].at[pl.ds(s * rows_per_tile, rows_per_tile)],
    )


def _sc_edge1(s_mat, src_glob, edst_pad, ald_tab, n1p):
    f = functools.partial(
        pl.kernel,
        out_type=jax.ShapeDtypeStruct((2, n1p, 72), F32),
        mesh=plsc.VectorSubcoreMesh(core_axis_name="c", subcore_axis_name="s"),
        compiler_params=pltpu.CompilerParams(
            needs_layout_passes=False, use_tc_tiling_on_sc=False),
        scratch_types=(
            [pltpu.VMEM((_B1K,), I32)] * 8
            + [pltpu.VMEM((_B1K, 80), F32)] * 2
            + [pltpu.VMEM((_B1K, 72), F32)] * 2
            + [pltpu.VMEM((_B1K, 8), F32)] * 2
            + [pltpu.VMEM((16, 8), F32)]
            + [pltpu.SemaphoreType.DMA] * 10
            + [
                pltpu.VMEM_SHARED((n1p, 8), F32),
                pltpu.VMEM_SHARED((n1p, 72), F32),
            ]
        ),
    )
    return f(_sc_edge1_body)(s_mat, src_glob, edst_pad, ald_tab)


def _sc_edge2_body(s2_hbm, a2_hbm, res1_hbm, esrc_hbm, edst_hbm, out_hbm,
                   a2_v, res_v, ald2_v,
                   sidx0, sidx1, sidx2, sidx3, didx0, didx1, didx2, didx3,
                   srow0, srow1, prod0, prod1,
                   isem0, isem1, isem2, isem3, gsem0, gsem1, ssem0, ssem1,
                   acc_sh):
    c = lax.axis_index("c")
    s = lax.axis_index("s")
    w = s * 2 + c
    sidx = [sidx0, sidx1, sidx2, sidx3]
    didx = [didx0, didx1, didx2, didx3]
    srow = [srow0, srow1]
    prod = [prod0, prod1]
    isem = [isem0, isem1, isem2, isem3]
    gsem = [gsem0, gsem1]
    ssem = [ssem0, ssem1]

    n2p = out_hbm.shape[1]
    rows_per_tile = n2p // 16
    epw = esrc_hbm.shape[0] // NWORK
    nch = epw // _B1K

    iota = _iota16()
    lane8 = _lane8()
    pat01 = _pat01()
    zero16 = jnp.zeros((16,), F32)
    zeros_i = iota * 0
    ones_i = zeros_i + 1
    col64 = zeros_i + 64
    excol = 64 + lane8

    pltpu.sync_copy(a2_hbm, a2_v)
    pltpu.sync_copy(res1_hbm, res_v)

    def bald(vi, carry):
        e = vi * 16
        r16 = res_v[pl.ds(e, 16)]
        ald2_v[pl.ds(e, 16)] = plsc.load_gather(a2_v, [r16, ones_i])
        return carry

    lax.fori_loop(0, res_v.shape[0] // 16, bald, 0)

    for prod_v in prod:
        def zrow(r, carry, prod_v=prod_v):
            for i in range(4):
                prod_v[r, pl.ds(i * 16, 16)] = zero16
            plsc.store_scatter(prod_v, [r + pat01, excol], zero16)
            return carry

        lax.fori_loop(0, _B1K, zrow, 0)
    pltpu.sync_copy(
        prod[0].at[pl.ds(0, rows_per_tile)],
        acc_sh.at[pl.ds(s * rows_per_tile, rows_per_tile)],
    )
    plsc.subcore_barrier()

    dclamp = ald2_v.shape[0] - 1

    def idx_descs(r, ci):
        base = w * epw + ci * _B1K
        return (
            pltpu.make_async_copy(
                esrc_hbm.at[pl.ds(base, _B1K)], sidx[r], isem[r]),
            pltpu.make_async_copy(
                edst_hbm.at[pl.ds(base, _B1K)], didx[r], isem[r]),
        )

    def gather_descs(s2, r):
        return (
            pltpu.make_async_copy(s2_hbm.at[sidx[r]], srow[s2], gsem[s2]),
        )

    def scatter_desc(s2, r):
        return pltpu.make_async_copy(prod[s2], acc_sh.at[didx[r]], ssem[s2])

    def compute(s2, b):
        sidx_v = sidx[b]
        didx_v = didx[b]
        srow_v = srow[s2]
        prod_w = prod[s2]

        @plsc.parallel_loop(0, _B1K // 16, unroll=2)
        def _(g):
            e0 = g * 16
            s16 = sidx_v[pl.ds(e0, 16)]
            d16 = didx_v[pl.ds(e0, 16)]
            d16 = jnp.minimum(d16, dclamp)
            als = plsc.load_gather(a2_v, [s16, zeros_i])
            ald = plsc.load_gather(ald2_v, [d16])
            a = als + ald
            ex = jnp.exp(jnp.maximum(a, LEAK * a))
            plsc.store_scatter(prod_w, [e0 + iota, col64], ex)
            for e in range(16):
                coef = _perm(ex, zeros_i + e)
                for i in range(4):
                    prod_w[e0 + e, pl.ds(i * 16, 16)] = (
                        srow_v[e0 + e, pl.ds(i * 16, 16)] * coef
                    )

    _edge_pipeline(nch, idx_descs, gather_descs, scatter_desc, compute)

    plsc.subcore_barrier()
    pltpu.sync_copy(
        acc_sh.at[pl.ds(s * rows_per_tile, rows_per_tile)],
        out_hbm.at[c].at[pl.ds(s * rows_per_tile, rows_per_tile)],
    )


def _sc_edge2(s2_mat, a2, res_n_id1, esrc_pad, edst_pad, n2p):
    f = functools.partial(
        pl.kernel,
        out_type=jax.ShapeDtypeStruct((2, n2p, 72), F32),
        mesh=plsc.VectorSubcoreMesh(core_axis_name="c", subcore_axis_name="s"),
        compiler_params=pltpu.CompilerParams(
            needs_layout_passes=False, use_tc_tiling_on_sc=False),
        scratch_types=(
            [
                pltpu.VMEM(a2.shape, F32),
                pltpu.VMEM(res_n_id1.shape, I32),
                pltpu.VMEM((res_n_id1.shape[0],), F32),
            ]
            + [pltpu.VMEM((_B1K,), I32)] * 8
            + [pltpu.VMEM((_B1K, 64), F32)] * 2
            + [pltpu.VMEM((_B1K, 72), F32)] * 2
            + [pltpu.SemaphoreType.DMA] * 8
            + [pltpu.VMEM_SHARED((n2p, 72), F32)]
        ),
    )
    return f(_sc_edge2_body)(s2_mat, a2, res_n_id1, esrc_pad, edst_pad)


def kernel(x, n_id0, res_n_id0, edge_src0, edge_dst0, res_n_id1, edge_src1,
           edge_dst1, W1, att_src1, att_dst1, b1, W2, att_src2, att_dst2, b2):
    n, d = x.shape
    n1 = res_n_id0.shape[0]
    n2 = res_n_id1.shape[0]
    e0 = edge_src0.shape[0]
    e1 = edge_src1.shape[0]
    h1, c1 = att_src1.shape

    e0p = ((e0 + NWORK * _B1K - 1) // (NWORK * _B1K)) * (NWORK * _B1K)
    e1p = ((e1 + NWORK * _B1K - 1) // (NWORK * _B1K)) * (NWORK * _B1K)
    n1p = n1 + 16
    n2p = n2 + 16

    w1r = W1.reshape(d, h1, c1)
    ws1 = (w1r * att_src1[None]).sum(-1)
    wd1 = (w1r * att_dst1[None]).sum(-1)
    w1e = jnp.concatenate([W1, ws1, wd1], axis=1)

    h2, c2 = att_src2.shape
    w2r = W2.reshape(h1 * c1, h2, c2)
    ws2 = (w2r * att_src2[None]).sum(-1)
    wd2 = (w2r * att_dst2[None]).sum(-1)
    w2e = jnp.concatenate([W2, ws2, wd2], axis=1)

    expmat = jnp.repeat(jnp.eye(h1, dtype=F32), c1, axis=1)

    pad0 = e0p - e0
    pad1 = e1p - e1
    esrc0_p = jnp.concatenate([edge_src0, jnp.zeros((pad0,), I32)])
    edst0_p = jnp.concatenate(
        [edge_dst0, n1 + (jnp.arange(pad0, dtype=I32) % 16)])
    esrc1_p = jnp.concatenate([edge_src1, jnp.zeros((pad1,), I32)])
    edst1_p = jnp.concatenate(
        [edge_dst1, n2 + (jnp.arange(pad1, dtype=I32) % 16)])

    s_mat = _tc_matmul(x, w1e, 1000)

    src_glob, ald_tab = _sc_prep(n_id0, esrc0_p, res_n_id0, s_mat)

    acc1 = _sc_edge1(s_mat, src_glob, edst0_p, ald_tab, n1p)

    s2_mat, a2 = _tc_combine1(acc1, b1.reshape(1, 64), w2e, expmat, n1)

    acc2 = _sc_edge2(s2_mat, a2, res_n_id1, esrc1_p, edst1_p, n2p)

    return _tc_final(acc2, b2.reshape(1, 64), n2, n2p)
